# Initial kernel scaffold; baseline (speedup 1.0000x reference)
import jax
import jax.numpy as jnp
from jax.experimental import pallas as pl


def kernel(x, target):
    raise NotImplementedError("write your pallas kernel here")


# speedup vs baseline: 2.2846x; 2.2846x over previous
import functools
import math

import jax
import jax.numpy as jnp
import numpy as np
from jax import lax
from jax.experimental import pallas as pl
from jax.experimental.pallas import tpu as pltpu

_PAD = 0
_V = 1000
_EPS = np.float32(0.1 / 998.0)
_CONF = np.float32(0.9)
_CROW = np.float32(998.0 * float(_EPS) * math.log(float(_EPS))
                   + 0.9 * math.log(0.9))

_ROWS_PER_BLK = 1024


def _tc_body(t_ref, x_ref, out_ref):
    i = pl.program_id(0)
    xb = x_ref[...]
    t = t_ref[...]
    live = t != _PAD
    col = lax.broadcasted_iota(jnp.int32, xb.shape, 1)
    w = jnp.where(col == t, _CONF, _EPS)
    w = jnp.where(col == _PAD, np.float32(0.0), w)
    w = jnp.where(live, w, np.float32(0.0))
    dot = jnp.sum(w * xb)
    nnz = jnp.sum(live.astype(jnp.float32))
    partial = nnz * _CROW - dot

    @pl.when(i == 0)
    def _():
        out_ref[0, 0] = np.float32(0.0)

    out_ref[0, 0] += partial


@jax.jit
def kernel(x, target):
    B, V = x.shape
    t = target.astype(jnp.int32).reshape(B, 1)
    grid = B // _ROWS_PER_BLK
    out = pl.pallas_call(
        _tc_body,
        grid=(grid,),
        in_specs=[
            pl.BlockSpec((_ROWS_PER_BLK, 1), lambda i: (i, 0)),
            pl.BlockSpec((_ROWS_PER_BLK, V), lambda i: (i, 0)),
        ],
        out_specs=pl.BlockSpec(memory_space=pltpu.SMEM),
        out_shape=jax.ShapeDtypeStruct((1, 1), jnp.float32),
    )(t, x)
    return out[0, 0]
